# baseline (speedup 1.0000x reference)
import functools

import jax
import jax.numpy as jnp
from jax import lax
from jax.experimental import pallas as pl
from jax.experimental.pallas import tpu as pltpu
from jax.experimental.pallas import tpu_sc as plsc

_DIM = 32
_NC, _NS = 2, 16
_NW = _NC * _NS
_SB = 8
_BB = 128

_mesh = plsc.VectorSubcoreMesh(
    core_axis_name="c", subcore_axis_name="s",
    num_cores=_NC, num_subcores=_NS)


def _transpose_slab(slab_v, rows_v, n_cols):
  d_lo = lax.iota(jnp.int32, 16)
  d_hi = d_lo + 16

  def body(c, carry):
    c_vec = jnp.full((16,), c, jnp.int32)
    v_lo = plsc.load_gather(slab_v, [d_lo, c_vec])
    v_hi = plsc.load_gather(slab_v, [d_hi, c_vec])
    rows_v[pl.ds(c * _DIM, 16)] = v_lo
    rows_v[pl.ds(c * _DIM + 16, 16)] = v_hi
    return carry

  lax.fori_loop(0, n_cols, body, 0)


@jax.jit
def _linearize_table(table_t, tail_flat):
  v_total = table_t.shape[1]
  n_full = v_total // _BB
  n_tail = v_total - n_full * _BB
  n_iter = (n_full + 1 + _NW - 1) // _NW

  @functools.partial(
      pl.kernel,
      out_type=jax.ShapeDtypeStruct((v_total * _DIM,), jnp.float32),
      mesh=_mesh,
      scratch_types=[
          pltpu.VMEM((_DIM, _BB), jnp.float32),
          pltpu.VMEM((_BB * _DIM,), jnp.float32),
      ],
      compiler_params=pltpu.CompilerParams(use_tc_tiling_on_sc=True,
                                           needs_layout_passes=False),
  )
  def linearize_kernel(tab_hbm, tail_hbm, out_hbm, slab_v, rows_v):
    wid = lax.axis_index("s") * _NC + lax.axis_index("c")

    def body(k, carry):
      j = k * _NW + wid

      @pl.when(j < n_full)
      def _():
        pltpu.sync_copy(tab_hbm.at[:, pl.ds(j * _BB, _BB)], slab_v)
        _transpose_slab(slab_v, rows_v, _BB)
        pltpu.sync_copy(rows_v, out_hbm.at[pl.ds(j * _BB * _DIM, _BB * _DIM)])

      @pl.when(j == n_full)
      def _():
        pltpu.sync_copy(tail_hbm, rows_v.at[pl.ds(0, n_tail * _DIM)])
        pltpu.sync_copy(rows_v.at[pl.ds(0, n_tail * _DIM)],
                        out_hbm.at[pl.ds(n_full * _BB * _DIM, n_tail * _DIM)])

      return carry

    lax.fori_loop(0, n_iter, body, 0)

  return linearize_kernel(table_t, tail_flat)


@functools.partial(jax.jit, static_argnums=(2, 3))
def _gather_rows(idx_t, table_lin, s_total, b_total):
  n_units = s_total // _SB

  @functools.partial(
      pl.kernel,
      out_type=jax.ShapeDtypeStruct((s_total, b_total, _DIM), jnp.float32),
      mesh=_mesh,
      scratch_types=[
          pltpu.VMEM((2, _SB, _BB), jnp.int32),
          pltpu.VMEM((2, _SB, _BB, _DIM), jnp.float32),
          pltpu.SemaphoreType.DMA((2,)),
          pltpu.SemaphoreType.DMA((2,)),
      ],
      compiler_params=pltpu.CompilerParams(use_tc_tiling_on_sc=False),
  )
  def gather_kernel(idx_hbm, table_hbm, out_hbm, idx_v, rows_v, gsem, ssem):
    wid = lax.axis_index("s") * _NC + lax.axis_index("c")
    b0 = wid * _BB

    def issue_gather(k, b):
      pltpu.sync_copy(idx_hbm.at[pl.ds(k * _SB, _SB), pl.ds(b0, _BB)],
                      idx_v.at[b])
      return [
          pltpu.async_copy(table_hbm.at[idx_v.at[b, si]], rows_v.at[b, si],
                           gsem.at[b])
          for si in range(_SB)
      ]

    def issue_store(k, b):
      return pltpu.async_copy(
          rows_v.at[b],
          out_hbm.at[pl.ds(k * _SB, _SB), pl.ds(b0, _BB), :],
          ssem.at[b])

    gathers = {0: issue_gather(0, 0)}
    stores = {}
    for k in range(n_units):
      b = k % 2
      if k + 1 < n_units:
        if k >= 1:
          stores.pop(k - 1).wait()
        gathers[k + 1] = issue_gather(k + 1, 1 - b)
      for h in gathers.pop(k):
        h.wait()
      stores[k] = issue_store(k, b)
    for k in sorted(stores):
      stores.pop(k).wait()

  return gather_kernel(idx_t, table_lin)


def kernel(i, table):
  b_total, s_total = i.shape
  n_full = (table.shape[0] // _BB) * _BB
  tail_flat = table[n_full:].reshape(-1)
  table_lin = _linearize_table(table.T, tail_flat).reshape(table.shape)
  out_t = _gather_rows(i.T, table_lin, s_total, b_total)
  return out_t.transpose(1, 0, 2)


# speedup vs baseline: 1.2182x; 1.2182x over previous
import functools

import jax
import jax.numpy as jnp
from jax import lax
from jax.experimental import pallas as pl
from jax.experimental.pallas import tpu as pltpu
from jax.experimental.pallas import tpu_sc as plsc

_DIM = 32
_NC, _NS = 2, 16
_NW = _NC * _NS
_SB = 8
_BB = 128

_mesh = plsc.VectorSubcoreMesh(
    core_axis_name="c", subcore_axis_name="s",
    num_cores=_NC, num_subcores=_NS)


_G = 7
_PER_W = 244
_GROUPS = [_G] * (_PER_W // _G) + ([_PER_W % _G] if _PER_W % _G else [])


def _transpose_group(slab_v, rows_v, slab_row0, rows_off0, n_slabs):
  d_lo = lax.iota(jnp.int32, 16)
  d_hi = d_lo + 16

  def step(cc, carry):
    for u in range(8):
      c = cc * 8 + u
      row0 = slab_row0 + lax.shift_right_logical(c, 7) * _DIM
      c_vec = jnp.full((16,), lax.bitwise_and(c, 127), jnp.int32)
      v_lo = plsc.load_gather(slab_v, [d_lo + row0, c_vec])
      v_hi = plsc.load_gather(slab_v, [d_hi + row0, c_vec])
      rows_v[pl.ds(rows_off0 + c * _DIM, 16)] = v_lo
      rows_v[pl.ds(rows_off0 + c * _DIM + 16, 16)] = v_hi
    return carry

  lax.fori_loop(0, n_slabs * 16, step, 0)


@jax.jit
def _linearize_table(table_t, tail_flat):
  v_total = table_t.shape[1]
  n_full = v_total // _BB
  n_tail = v_total - n_full * _BB

  @functools.partial(
      pl.kernel,
      out_type=jax.ShapeDtypeStruct((v_total * _DIM,), jnp.float32),
      mesh=_mesh,
      scratch_types=[
          pltpu.VMEM((2 * _G * _DIM, _BB), jnp.float32),
          pltpu.VMEM((2 * _G * _BB * _DIM,), jnp.float32),
          pltpu.SemaphoreType.DMA((2,)),
          pltpu.SemaphoreType.DMA((2,)),
      ],
      compiler_params=pltpu.CompilerParams(use_tc_tiling_on_sc=True,
                                           needs_layout_passes=False),
  )
  def linearize_kernel(tab_hbm, tail_hbm, out_hbm, slab_v, rows_v, rsem, ssem):
    wid = lax.axis_index("s") * _NC + lax.axis_index("c")
    base_j = wid * _PER_W
    starts = [sum(_GROUPS[:g]) for g in range(len(_GROUPS))]

    def issue_reads(g, b):
      c0 = (base_j + starts[g]) * _BB
      return [
          pltpu.async_copy(
              tab_hbm.at[:, pl.ds(c0 + jj * _BB, _BB)],
              slab_v.at[pl.ds((b * _G + jj) * _DIM, _DIM), :], rsem.at[b])
          for jj in range(_GROUPS[g])
      ]

    def issue_store(g, b):
      o0 = (base_j + starts[g]) * _BB * _DIM
      n = _GROUPS[g] * _BB * _DIM
      return pltpu.async_copy(rows_v.at[pl.ds(b * _G * _BB * _DIM, n)],
                              out_hbm.at[pl.ds(o0, n)], ssem.at[b])

    n_groups = len(_GROUPS)
    reads = {0: issue_reads(0, 0)}
    stores = {}
    for g in range(n_groups):
      b = g % 2
      for h in reads.pop(g):
        h.wait()
      if g + 1 < n_groups:
        reads[g + 1] = issue_reads(g + 1, 1 - b)
      if g >= 2:
        stores.pop(g - 2).wait()
      _transpose_group(slab_v, rows_v, b * _G * _DIM, b * _G * _BB * _DIM,
                       _GROUPS[g])
      stores[g] = issue_store(g, b)
    for g in sorted(stores):
      stores.pop(g).wait()

    @pl.when(wid < n_full - _NW * _PER_W)
    def _():
      j = _NW * _PER_W + wid
      pltpu.sync_copy(tab_hbm.at[:, pl.ds(j * _BB, _BB)],
                      slab_v.at[pl.ds(0, _DIM), :])
      _transpose_group(slab_v, rows_v, 0, 0, 1)
      pltpu.sync_copy(rows_v.at[pl.ds(0, _BB * _DIM)],
                      out_hbm.at[pl.ds(j * _BB * _DIM, _BB * _DIM)])

    @pl.when(wid == n_full - _NW * _PER_W)
    def _():
      pltpu.sync_copy(tail_hbm, rows_v.at[pl.ds(0, n_tail * _DIM)])
      pltpu.sync_copy(rows_v.at[pl.ds(0, n_tail * _DIM)],
                      out_hbm.at[pl.ds(n_full * _BB * _DIM, n_tail * _DIM)])

  return linearize_kernel(table_t, tail_flat)


@functools.partial(jax.jit, static_argnums=(2, 3))
def _gather_rows(idx_t, table_lin, s_total, b_total):
  n_units = s_total // _SB

  @functools.partial(
      pl.kernel,
      out_type=jax.ShapeDtypeStruct((s_total, b_total, _DIM), jnp.float32),
      mesh=_mesh,
      scratch_types=[
          pltpu.VMEM((2, _SB, _BB), jnp.int32),
          pltpu.VMEM((2, _SB, _BB, _DIM), jnp.float32),
          pltpu.SemaphoreType.DMA((2,)),
          pltpu.SemaphoreType.DMA((2,)),
      ],
      compiler_params=pltpu.CompilerParams(use_tc_tiling_on_sc=False),
  )
  def gather_kernel(idx_hbm, table_hbm, out_hbm, idx_v, rows_v, gsem, ssem):
    wid = lax.axis_index("s") * _NC + lax.axis_index("c")
    b0 = wid * _BB

    def issue_gather(k, b):
      pltpu.sync_copy(idx_hbm.at[pl.ds(k * _SB, _SB), pl.ds(b0, _BB)],
                      idx_v.at[b])
      return [
          pltpu.async_copy(table_hbm.at[idx_v.at[b, si]], rows_v.at[b, si],
                           gsem.at[b])
          for si in range(_SB)
      ]

    def issue_store(k, b):
      return pltpu.async_copy(
          rows_v.at[b],
          out_hbm.at[pl.ds(k * _SB, _SB), pl.ds(b0, _BB), :],
          ssem.at[b])

    gathers = {0: issue_gather(0, 0)}
    stores = {}
    for k in range(n_units):
      b = k % 2
      if k + 1 < n_units:
        if k >= 1:
          stores.pop(k - 1).wait()
        gathers[k + 1] = issue_gather(k + 1, 1 - b)
      for h in gathers.pop(k):
        h.wait()
      stores[k] = issue_store(k, b)
    for k in sorted(stores):
      stores.pop(k).wait()

  return gather_kernel(idx_t, table_lin)


def kernel(i, table):
  b_total, s_total = i.shape
  n_full = (table.shape[0] // _BB) * _BB
  tail_flat = table[n_full:].reshape(-1)
  table_lin = _linearize_table(table.T, tail_flat).reshape(table.shape)
  out_t = _gather_rows(i.T, table_lin, s_total, b_total)
  return out_t.transpose(1, 0, 2)


# speedup vs baseline: 1.4875x; 1.2210x over previous
import functools

import jax
import jax.numpy as jnp
from jax import lax
from jax.experimental import pallas as pl
from jax.experimental.pallas import tpu as pltpu
from jax.experimental.pallas import tpu_sc as plsc

_DIM = 32
_NC, _NS = 2, 16
_NW = _NC * _NS
_SB = 8
_BB = 128

_mesh = plsc.VectorSubcoreMesh(
    core_axis_name="c", subcore_axis_name="s",
    num_cores=_NC, num_subcores=_NS)


_G = 7
_PER_W = 244
_GROUPS = [_G] * (_PER_W // _G) + ([_PER_W % _G] if _PER_W % _G else [])


def _transpose_group(slab_v, rows_v, slab_row0, rows_off0, n_slabs):
  d_lo = lax.iota(jnp.int32, 16)
  d_hi = d_lo + 16

  @plsc.parallel_loop(0, n_slabs * 128, unroll=8)
  def _(c):
    row0 = slab_row0 + lax.shift_right_logical(c, 7) * _DIM
    c_vec = jnp.full((16,), lax.bitwise_and(c, 127), jnp.int32)
    v_lo = plsc.load_gather(slab_v, [d_lo + row0, c_vec])
    v_hi = plsc.load_gather(slab_v, [d_hi + row0, c_vec])
    rows_v[pl.ds(rows_off0 + c * _DIM, 16)] = v_lo
    rows_v[pl.ds(rows_off0 + c * _DIM + 16, 16)] = v_hi


@jax.jit
def _linearize_table(table_t, tail_flat):
  v_total = table_t.shape[1]
  n_full = v_total // _BB
  n_tail = v_total - n_full * _BB

  @functools.partial(
      pl.kernel,
      out_type=jax.ShapeDtypeStruct((v_total * _DIM,), jnp.float32),
      mesh=_mesh,
      scratch_types=[
          pltpu.VMEM((2 * _G * _DIM, _BB), jnp.float32),
          pltpu.VMEM((2 * _G * _BB * _DIM,), jnp.float32),
          pltpu.SemaphoreType.DMA((2,)),
          pltpu.SemaphoreType.DMA((2,)),
      ],
      compiler_params=pltpu.CompilerParams(use_tc_tiling_on_sc=True,
                                           needs_layout_passes=False),
  )
  def linearize_kernel(tab_hbm, tail_hbm, out_hbm, slab_v, rows_v, rsem, ssem):
    wid = lax.axis_index("s") * _NC + lax.axis_index("c")
    base_j = wid * _PER_W
    starts = [sum(_GROUPS[:g]) for g in range(len(_GROUPS))]

    def issue_reads(g, b):
      c0 = (base_j + starts[g]) * _BB
      return [
          pltpu.async_copy(
              tab_hbm.at[:, pl.ds(c0 + jj * _BB, _BB)],
              slab_v.at[pl.ds((b * _G + jj) * _DIM, _DIM), :], rsem.at[b])
          for jj in range(_GROUPS[g])
      ]

    def issue_store(g, b):
      o0 = (base_j + starts[g]) * _BB * _DIM
      n = _GROUPS[g] * _BB * _DIM
      return pltpu.async_copy(rows_v.at[pl.ds(b * _G * _BB * _DIM, n)],
                              out_hbm.at[pl.ds(o0, n)], ssem.at[b])

    n_groups = len(_GROUPS)
    reads = {0: issue_reads(0, 0)}
    stores = {}
    for g in range(n_groups):
      b = g % 2
      for h in reads.pop(g):
        h.wait()
      if g + 1 < n_groups:
        reads[g + 1] = issue_reads(g + 1, 1 - b)
      if g >= 2:
        stores.pop(g - 2).wait()
      _transpose_group(slab_v, rows_v, b * _G * _DIM, b * _G * _BB * _DIM,
                       _GROUPS[g])
      stores[g] = issue_store(g, b)
    for g in sorted(stores):
      stores.pop(g).wait()

    @pl.when(wid < n_full - _NW * _PER_W)
    def _():
      j = _NW * _PER_W + wid
      pltpu.sync_copy(tab_hbm.at[:, pl.ds(j * _BB, _BB)],
                      slab_v.at[pl.ds(0, _DIM), :])
      _transpose_group(slab_v, rows_v, 0, 0, 1)
      pltpu.sync_copy(rows_v.at[pl.ds(0, _BB * _DIM)],
                      out_hbm.at[pl.ds(j * _BB * _DIM, _BB * _DIM)])

    @pl.when(wid == n_full - _NW * _PER_W)
    def _():
      pltpu.sync_copy(tail_hbm, rows_v.at[pl.ds(0, n_tail * _DIM)])
      pltpu.sync_copy(rows_v.at[pl.ds(0, n_tail * _DIM)],
                      out_hbm.at[pl.ds(n_full * _BB * _DIM, n_tail * _DIM)])

  return linearize_kernel(table_t, tail_flat)


@functools.partial(jax.jit, static_argnums=(2, 3))
def _gather_rows(idx_t, table_lin, s_total, b_total):
  n_units = s_total // _SB

  @functools.partial(
      pl.kernel,
      out_type=jax.ShapeDtypeStruct((s_total, b_total, _DIM), jnp.float32),
      mesh=_mesh,
      scratch_types=[
          pltpu.VMEM((2, _SB, _BB), jnp.int32),
          pltpu.VMEM((2, _SB, _BB, _DIM), jnp.float32),
          pltpu.SemaphoreType.DMA((2,)),
          pltpu.SemaphoreType.DMA((2,)),
      ],
      compiler_params=pltpu.CompilerParams(use_tc_tiling_on_sc=False),
  )
  def gather_kernel(idx_hbm, table_hbm, out_hbm, idx_v, rows_v, gsem, ssem):
    wid = lax.axis_index("s") * _NC + lax.axis_index("c")
    b0 = wid * _BB

    def issue_gather(k, b):
      pltpu.sync_copy(idx_hbm.at[pl.ds(k * _SB, _SB), pl.ds(b0, _BB)],
                      idx_v.at[b])
      return [
          pltpu.async_copy(table_hbm.at[idx_v.at[b, si]], rows_v.at[b, si],
                           gsem.at[b])
          for si in range(_SB)
      ]

    def issue_store(k, b):
      return pltpu.async_copy(
          rows_v.at[b],
          out_hbm.at[pl.ds(k * _SB, _SB), pl.ds(b0, _BB), :],
          ssem.at[b])

    gathers = {0: issue_gather(0, 0)}
    stores = {}
    for k in range(n_units):
      b = k % 2
      if k + 1 < n_units:
        if k >= 1:
          stores.pop(k - 1).wait()
        gathers[k + 1] = issue_gather(k + 1, 1 - b)
      for h in gathers.pop(k):
        h.wait()
      stores[k] = issue_store(k, b)
    for k in sorted(stores):
      stores.pop(k).wait()

  return gather_kernel(idx_t, table_lin)


def kernel(i, table):
  b_total, s_total = i.shape
  n_full = (table.shape[0] // _BB) * _BB
  tail_flat = table[n_full:].reshape(-1)
  table_lin = _linearize_table(table.T, tail_flat).reshape(table.shape)
  out_t = _gather_rows(i.T, table_lin, s_total, b_total)
  return out_t.transpose(1, 0, 2)


# speedup vs baseline: 1.5032x; 1.0106x over previous
import functools

import jax
import jax.numpy as jnp
from jax import lax
from jax.experimental import pallas as pl
from jax.experimental.pallas import tpu as pltpu
from jax.experimental.pallas import tpu_sc as plsc

_DIM = 32
_NC, _NS = 2, 16
_NW = _NC * _NS
_SB = 8
_BB = 128

_mesh = plsc.VectorSubcoreMesh(
    core_axis_name="c", subcore_axis_name="s",
    num_cores=_NC, num_subcores=_NS)


_G = 7
_PER_W = 244
_GROUPS = [_G] * (_PER_W // _G) + ([_PER_W % _G] if _PER_W % _G else [])


def _transpose_group(slab_v, rows_v, slab_row0, rows_off0, n_slabs):
  d_lo = lax.iota(jnp.int32, 16)
  d_hi = d_lo + 16

  @plsc.parallel_loop(0, n_slabs * 128, unroll=16)
  def _(c):
    row0 = slab_row0 + lax.shift_right_logical(c, 7) * _DIM
    c_vec = jnp.full((16,), lax.bitwise_and(c, 127), jnp.int32)
    v_lo = plsc.load_gather(slab_v, [d_lo + row0, c_vec])
    v_hi = plsc.load_gather(slab_v, [d_hi + row0, c_vec])
    rows_v[pl.ds(rows_off0 + c * _DIM, 16)] = v_lo
    rows_v[pl.ds(rows_off0 + c * _DIM + 16, 16)] = v_hi


@jax.jit
def _linearize_table(table_t, tail_flat):
  v_total = table_t.shape[1]
  n_full = v_total // _BB
  n_tail = v_total - n_full * _BB

  @functools.partial(
      pl.kernel,
      out_type=jax.ShapeDtypeStruct((v_total * _DIM,), jnp.float32),
      mesh=_mesh,
      scratch_types=[
          pltpu.VMEM((2 * _G * _DIM, _BB), jnp.float32),
          pltpu.VMEM((2 * _G * _BB * _DIM,), jnp.float32),
          pltpu.SemaphoreType.DMA((2,)),
          pltpu.SemaphoreType.DMA((2,)),
      ],
      compiler_params=pltpu.CompilerParams(use_tc_tiling_on_sc=True,
                                           needs_layout_passes=False),
  )
  def linearize_kernel(tab_hbm, tail_hbm, out_hbm, slab_v, rows_v, rsem, ssem):
    wid = lax.axis_index("s") * _NC + lax.axis_index("c")
    base_j = wid * _PER_W
    starts = [sum(_GROUPS[:g]) for g in range(len(_GROUPS))]

    def issue_reads(g, b):
      c0 = (base_j + starts[g]) * _BB
      return [
          pltpu.async_copy(
              tab_hbm.at[:, pl.ds(c0 + jj * _BB, _BB)],
              slab_v.at[pl.ds((b * _G + jj) * _DIM, _DIM), :], rsem.at[b])
          for jj in range(_GROUPS[g])
      ]

    def issue_store(g, b):
      o0 = (base_j + starts[g]) * _BB * _DIM
      n = _GROUPS[g] * _BB * _DIM
      return pltpu.async_copy(rows_v.at[pl.ds(b * _G * _BB * _DIM, n)],
                              out_hbm.at[pl.ds(o0, n)], ssem.at[b])

    n_groups = len(_GROUPS)
    reads = {0: issue_reads(0, 0)}
    stores = {}
    for g in range(n_groups):
      b = g % 2
      for h in reads.pop(g):
        h.wait()
      if g + 1 < n_groups:
        reads[g + 1] = issue_reads(g + 1, 1 - b)
      if g >= 2:
        stores.pop(g - 2).wait()
      _transpose_group(slab_v, rows_v, b * _G * _DIM, b * _G * _BB * _DIM,
                       _GROUPS[g])
      stores[g] = issue_store(g, b)
    for g in sorted(stores):
      stores.pop(g).wait()

    @pl.when(wid < n_full - _NW * _PER_W)
    def _():
      j = _NW * _PER_W + wid
      pltpu.sync_copy(tab_hbm.at[:, pl.ds(j * _BB, _BB)],
                      slab_v.at[pl.ds(0, _DIM), :])
      _transpose_group(slab_v, rows_v, 0, 0, 1)
      pltpu.sync_copy(rows_v.at[pl.ds(0, _BB * _DIM)],
                      out_hbm.at[pl.ds(j * _BB * _DIM, _BB * _DIM)])

    @pl.when(wid == n_full - _NW * _PER_W)
    def _():
      pltpu.sync_copy(tail_hbm, rows_v.at[pl.ds(0, n_tail * _DIM)])
      pltpu.sync_copy(rows_v.at[pl.ds(0, n_tail * _DIM)],
                      out_hbm.at[pl.ds(n_full * _BB * _DIM, n_tail * _DIM)])

  return linearize_kernel(table_t, tail_flat)


@functools.partial(jax.jit, static_argnums=(2, 3))
def _gather_rows(idx_t, table_lin, s_total, b_total):
  n_units = s_total // _SB

  @functools.partial(
      pl.kernel,
      out_type=jax.ShapeDtypeStruct((s_total, b_total, _DIM), jnp.float32),
      mesh=_mesh,
      scratch_types=[
          pltpu.VMEM((2, _SB, _BB), jnp.int32),
          pltpu.VMEM((2, _SB, _BB, _DIM), jnp.float32),
          pltpu.SemaphoreType.DMA((2,)),
          pltpu.SemaphoreType.DMA((2,)),
      ],
      compiler_params=pltpu.CompilerParams(use_tc_tiling_on_sc=False),
  )
  def gather_kernel(idx_hbm, table_hbm, out_hbm, idx_v, rows_v, gsem, ssem):
    wid = lax.axis_index("s") * _NC + lax.axis_index("c")
    b0 = wid * _BB

    def issue_gather(k, b):
      pltpu.sync_copy(idx_hbm.at[pl.ds(k * _SB, _SB), pl.ds(b0, _BB)],
                      idx_v.at[b])
      return [
          pltpu.async_copy(table_hbm.at[idx_v.at[b, si]], rows_v.at[b, si],
                           gsem.at[b])
          for si in range(_SB)
      ]

    def issue_store(k, b):
      return pltpu.async_copy(
          rows_v.at[b],
          out_hbm.at[pl.ds(k * _SB, _SB), pl.ds(b0, _BB), :],
          ssem.at[b])

    gathers = {0: issue_gather(0, 0)}
    stores = {}
    for k in range(n_units):
      b = k % 2
      if k + 1 < n_units:
        if k >= 1:
          stores.pop(k - 1).wait()
        gathers[k + 1] = issue_gather(k + 1, 1 - b)
      for h in gathers.pop(k):
        h.wait()
      stores[k] = issue_store(k, b)
    for k in sorted(stores):
      stores.pop(k).wait()

  return gather_kernel(idx_t, table_lin)


def kernel(i, table):
  b_total, s_total = i.shape
  n_full = (table.shape[0] // _BB) * _BB
  tail_flat = table[n_full:].reshape(-1)
  table_lin = _linearize_table(table.T, tail_flat).reshape(table.shape)
  out_t = _gather_rows(i.T, table_lin, s_total, b_total)
  return out_t.transpose(1, 0, 2)
